# Initial kernel scaffold; baseline (speedup 1.0000x reference)
import jax
import jax.numpy as jnp
from jax.experimental import pallas as pl


def kernel(x, edge_index, edge_type, w1, root1, bias1, comp2, basis2, root2, bias2):
    raise NotImplementedError("write your pallas kernel here")


# speedup vs baseline: 3.3232x; 3.3232x over previous
import functools

import jax
import jax.numpy as jnp
from jax import lax
from jax.experimental import pallas as pl
from jax.experimental.pallas import tpu as pltpu
from jax.experimental.pallas import tpu_sc as plsc

_N = 10000
_E = 320000
_R = 8
_D = 128
_NB = 4
_NC = 2
_NS = 16
_NT = _NC * _NS
_NPT = 320
_NPAD = _NT * _NPT
_CH = 128
_RB = 2560
_NEG = -3.0e38
_NEGH = -1.0e38


def _mm_body(x_ref, w_ref, b_ref, o_ref):
    r = pl.program_id(0)
    y = jnp.dot(x_ref[...], w_ref[0], preferred_element_type=jnp.float32)
    o_ref[0] = y + jnp.where(r == _R, b_ref[...], 0.0)


def _mm(xp, W, bias):
    out = pl.pallas_call(
        _mm_body,
        grid=(_R + 1, _NPAD // _RB),
        in_specs=[
            pl.BlockSpec((_RB, _D), lambda r, i: (i, 0)),
            pl.BlockSpec((1, _D, _D), lambda r, i: (r, 0, 0)),
            pl.BlockSpec((1, _D), lambda r, i: (0, 0)),
        ],
        out_specs=pl.BlockSpec((1, _RB, _D), lambda r, i: (r, i, 0)),
        out_shape=jax.ShapeDtypeStruct((_R + 1, _NPAD, _D), jnp.float32),
    )(xp, W, bias.reshape(1, _D))
    return out.reshape((_R + 1) * _NPAD, _D)


def _agg_body(do_relu, table, gidx, dl, bounds, out,
              bnd_v, idx_v, dl_v, rows_v, acc, sacc, bnd_s, sem):
    c = lax.axis_index("c")
    s = lax.axis_index("s")
    T = s * _NC + c

    pltpu.sync_copy(bounds.at[T], bnd_v)
    bv = bnd_v[...]
    for i in range(9):
        bnd_s[i] = bv[i]

    zeros16 = jnp.zeros((16,), jnp.float32)
    negs16 = jnp.full((16,), _NEG, jnp.float32)
    iota16 = lax.iota(jnp.int32, 16)

    def zrow(row, carry):
        for sl in range(8):
            sacc[row, pl.ds(sl * 16, 16)] = zeros16
        return carry

    lax.fori_loop(0, _NPT, zrow, 0)

    def rel_body(r, carry):
        start = bnd_s[r]
        end = bnd_s[r + 1]
        astart = (start // 8) * 8
        nch = (end - astart + _CH - 1) // _CH

        def irow(row, cc):
            for sl in range(8):
                acc[row, pl.ds(sl * 16, 16)] = negs16
            return cc

        lax.fori_loop(0, _NPT + 1, irow, 0)

        def chunk(ci, cc):
            base = astart + ci * _CH
            pltpu.sync_copy(gidx.at[pl.ds(base, _CH)], idx_v)
            pltpu.sync_copy(dl.at[pl.ds(base, _CH)], dl_v)
            pltpu.async_copy(table.at[idx_v], rows_v, sem).wait()

            def grp(g, gc):
                off = g * 16
                pos = base + off + iota16
                dlv = dl_v[pl.ds(off, 16)]
                ok = (pos >= start) & (pos < end)
                dlm = jnp.where(ok, dlv, _NPT)
                for lane in range(16):
                    row = dlm[lane]
                    e = off + lane
                    for sl in range(8):
                        cur = acc[row, pl.ds(sl * 16, 16)]
                        msg = rows_v[e, pl.ds(sl * 16, 16)]
                        acc[row, pl.ds(sl * 16, 16)] = jnp.maximum(cur, msg)
                return gc

            lax.fori_loop(0, _CH // 16, grp, 0)
            return cc

        lax.fori_loop(0, nch, chunk, 0)

        def frow(row, cc):
            for sl in range(8):
                a = acc[row, pl.ds(sl * 16, 16)]
                prev = sacc[row, pl.ds(sl * 16, 16)]
                sacc[row, pl.ds(sl * 16, 16)] = prev + jnp.where(a > _NEGH, a, 0.0)
            return cc

        lax.fori_loop(0, _NPT, frow, 0)
        return carry

    lax.fori_loop(0, _R, rel_body, 0)

    pltpu.sync_copy(table.at[pl.ds(_R * _NPAD + T * _NPT, _NPT)],
                    acc.at[pl.ds(0, _NPT)])

    def orow(row, cc):
        for sl in range(8):
            v = sacc[row, pl.ds(sl * 16, 16)] + acc[row, pl.ds(sl * 16, 16)]
            if do_relu:
                v = jnp.maximum(v, 0.0)
            sacc[row, pl.ds(sl * 16, 16)] = v
        return cc

    lax.fori_loop(0, _NPT, orow, 0)
    pltpu.sync_copy(sacc, out.at[pl.ds(T * _NPT, _NPT)])


def _agg(do_relu):
    mesh = plsc.VectorSubcoreMesh(core_axis_name="c", subcore_axis_name="s")
    return pl.kernel(
        functools.partial(_agg_body, do_relu),
        out_type=jax.ShapeDtypeStruct((_NPAD, _D), jnp.float32),
        mesh=mesh,
        scratch_types=[
            pltpu.VMEM((16,), jnp.int32),
            pltpu.VMEM((_CH,), jnp.int32),
            pltpu.VMEM((_CH,), jnp.int32),
            pltpu.VMEM((_CH, _D), jnp.float32),
            pltpu.VMEM((_NPT + 1, _D), jnp.float32),
            pltpu.VMEM((_NPT, _D), jnp.float32),
            pltpu.SMEM((16,), jnp.int32),
            pltpu.SemaphoreType.DMA,
        ],
    )


def kernel(x, edge_index, edge_type, w1, root1, bias1, comp2, basis2, root2, bias2):
    src = edge_index[0]
    dst = edge_index[1]
    et = edge_type

    W1 = jnp.zeros((_R, _NB, _D // _NB, _NB, _D // _NB), jnp.float32)
    for b in range(_NB):
        W1 = W1.at[:, b, :, b, :].set(w1[:, b])
    W1 = W1.reshape(_R, _D, _D)
    W1a = jnp.concatenate([W1, root1[None]], axis=0)
    W2 = jnp.einsum('rb,bio->rio', comp2, basis2)
    W2a = jnp.concatenate([W2, root2[None]], axis=0)

    tid = dst // _NPT
    binid = tid * _R + et
    gidx = et * _NPAD + src
    dloc = dst - tid * _NPT
    binid_s, gidx_s, dl_s = lax.sort((binid, gidx, dloc), num_keys=1)
    bidx = jnp.arange(_NT)[:, None] * _R + jnp.minimum(jnp.arange(16)[None, :], _R)
    bounds = jnp.searchsorted(binid_s, bidx).astype(jnp.int32)
    gidx_p = jnp.concatenate([gidx_s, jnp.zeros((_CH,), jnp.int32)])
    dl_p = jnp.concatenate([dl_s, jnp.full((_CH,), _NPT, jnp.int32)])

    xpad = jnp.zeros((_NPAD, _D), jnp.float32).at[:_N].set(x)
    table1 = _mm(xpad, W1a, bias1)
    h = _agg(True)(table1, gidx_p, dl_p, bounds)
    table2 = _mm(h, W2a, bias2)
    out = _agg(False)(table2, gidx_p, dl_p, bounds)
    return out[:_N]


# speedup vs baseline: 4.0776x; 1.2270x over previous
import functools

import jax
import jax.numpy as jnp
from jax import lax
from jax.experimental import pallas as pl
from jax.experimental.pallas import tpu as pltpu
from jax.experimental.pallas import tpu_sc as plsc

_N = 10000
_E = 320000
_R = 8
_D = 128
_NB = 4
_NC = 2
_NS = 16
_NT = _NC * _NS
_NPT = 320
_NPAD = _NT * _NPT
_CH = 128
_RB = 2560
_NEG = -3.0e38
_NEGH = -1.0e38


def _mm_body(x_ref, w_ref, b_ref, o_ref):
    r = pl.program_id(0)
    y = jnp.dot(x_ref[...], w_ref[0], preferred_element_type=jnp.float32)
    o_ref[0] = y + jnp.where(r == _R, b_ref[...], 0.0)


def _mm(xp, W, bias):
    out = pl.pallas_call(
        _mm_body,
        grid=(_R + 1, _NPAD // _RB),
        in_specs=[
            pl.BlockSpec((_RB, _D), lambda r, i: (i, 0)),
            pl.BlockSpec((1, _D, _D), lambda r, i: (r, 0, 0)),
            pl.BlockSpec((1, _D), lambda r, i: (0, 0)),
        ],
        out_specs=pl.BlockSpec((1, _RB, _D), lambda r, i: (r, i, 0)),
        out_shape=jax.ShapeDtypeStruct((_R + 1, _NPAD, _D), jnp.float32),
    )(xp, W, bias.reshape(1, _D))
    return out.reshape((_R + 1) * _NPAD, _D)


def _agg_body(do_relu, table, edata, bounds, out,
              bnd_v, ebuf, rows, acc, sacc, bnd_s, semE, semR0, semR1):
    ci_ = lax.axis_index("c")
    si_ = lax.axis_index("s")
    T = si_ * _NC + ci_

    pltpu.sync_copy(bounds.at[T], bnd_v)
    bv = bnd_v[...]
    for i in range(9):
        bnd_s[i] = bv[i]

    pltpu.sync_copy(table.at[pl.ds(_R * _NPAD + T * _NPT, _NPT)], sacc)

    negs16 = jnp.full((16,), _NEG, jnp.float32)
    iota16 = lax.iota(jnp.int32, 16)

    def irow(row, cc):
        for sl in range(8):
            acc[row, pl.ds(sl * 16, 16)] = negs16
        return cc

    lax.fori_loop(0, _NPT + 1, irow, 0)

    def rel_body(r, carry):
        start = bnd_s[r]
        end = bnd_s[r + 1]
        astart = (start // _CH) * _CH
        nch = (end - astart + _CH - 1) // _CH

        def esrc(ci):
            return edata.at[:, pl.ds(astart + ci * _CH, _CH)]

        def estart(ci):
            pltpu.async_copy(esrc(ci), ebuf.at[lax.rem(ci, 3)], semE)

        def ewait(ci):
            pltpu.make_async_copy(esrc(ci), ebuf.at[lax.rem(ci, 3)], semE).wait()

        def gstart(ci):
            e3 = lax.rem(ci, 3)
            p = lax.rem(ci, 2)

            @pl.when(p == 0)
            def _():
                pltpu.async_copy(table.at[ebuf.at[e3, 0]], rows.at[0], semR0)

            @pl.when(p == 1)
            def _():
                pltpu.async_copy(table.at[ebuf.at[e3, 0]], rows.at[1], semR1)

        def gwait(ci):
            e3 = lax.rem(ci, 3)
            p = lax.rem(ci, 2)

            @pl.when(p == 0)
            def _():
                pltpu.make_async_copy(table.at[ebuf.at[e3, 0]], rows.at[0], semR0).wait()

            @pl.when(p == 1)
            def _():
                pltpu.make_async_copy(table.at[ebuf.at[e3, 0]], rows.at[1], semR1).wait()

        @pl.when(nch > 0)
        def _():
            estart(0)
            ewait(0)
            gstart(0)

        @pl.when(nch > 1)
        def _():
            estart(1)

        def chunk(ci, cc):
            p = lax.rem(ci, 2)
            e3 = lax.rem(ci, 3)

            @pl.when(ci + 1 < nch)
            def _():
                ewait(ci + 1)
                gstart(ci + 1)

            @pl.when(ci + 2 < nch)
            def _():
                estart(ci + 2)

            gwait(ci)
            base = astart + ci * _CH

            def grp(g, gc):
                off = g * 16
                pos = base + off + iota16
                dlv = ebuf[e3, 1, pl.ds(off, 16)]
                ok = (pos >= start) & (pos < end)
                dlm = jnp.where(ok, dlv, _NPT)
                for lane in range(16):
                    row = dlm[lane]
                    e = off + lane
                    for sl in range(8):
                        cur = acc[row, pl.ds(sl * 16, 16)]
                        msg = rows[p, e, pl.ds(sl * 16, 16)]
                        acc[row, pl.ds(sl * 16, 16)] = jnp.maximum(cur, msg)
                return gc

            lax.fori_loop(0, _CH // 16, grp, 0)
            return cc

        lax.fori_loop(0, nch, chunk, 0)

        def frow(row, cc):
            for sl in range(8):
                a = acc[row, pl.ds(sl * 16, 16)]
                prev = sacc[row, pl.ds(sl * 16, 16)]
                sacc[row, pl.ds(sl * 16, 16)] = prev + jnp.where(a > _NEGH, a, 0.0)
                acc[row, pl.ds(sl * 16, 16)] = negs16
            return cc

        lax.fori_loop(0, _NPT, frow, 0)
        return carry

    lax.fori_loop(0, _R, rel_body, 0)

    if do_relu:
        def rrow(row, cc):
            for sl in range(8):
                v = sacc[row, pl.ds(sl * 16, 16)]
                sacc[row, pl.ds(sl * 16, 16)] = jnp.maximum(v, 0.0)
            return cc

        lax.fori_loop(0, _NPT, rrow, 0)

    pltpu.sync_copy(sacc, out.at[pl.ds(T * _NPT, _NPT)])


def _agg(do_relu):
    mesh = plsc.VectorSubcoreMesh(core_axis_name="c", subcore_axis_name="s")
    return pl.kernel(
        functools.partial(_agg_body, do_relu),
        out_type=jax.ShapeDtypeStruct((_NPAD, _D), jnp.float32),
        mesh=mesh,
        scratch_types=[
            pltpu.VMEM((16,), jnp.int32),
            pltpu.VMEM((3, 2, _CH), jnp.int32),
            pltpu.VMEM((2, _CH, _D), jnp.float32),
            pltpu.VMEM((_NPT + 1, _D), jnp.float32),
            pltpu.VMEM((_NPT, _D), jnp.float32),
            pltpu.SMEM((16,), jnp.int32),
            pltpu.SemaphoreType.DMA,
            pltpu.SemaphoreType.DMA,
            pltpu.SemaphoreType.DMA,
        ],
    )


def kernel(x, edge_index, edge_type, w1, root1, bias1, comp2, basis2, root2, bias2):
    src = edge_index[0]
    dst = edge_index[1]
    et = edge_type

    W1 = jnp.zeros((_R, _NB, _D // _NB, _NB, _D // _NB), jnp.float32)
    for b in range(_NB):
        W1 = W1.at[:, b, :, b, :].set(w1[:, b])
    W1 = W1.reshape(_R, _D, _D)
    W1a = jnp.concatenate([W1, root1[None]], axis=0)
    W2 = jnp.einsum('rb,bio->rio', comp2, basis2)
    W2a = jnp.concatenate([W2, root2[None]], axis=0)

    tid = dst // _NPT
    binid = tid * _R + et
    gidx = et * _NPAD + src
    dloc = dst - tid * _NPT
    binid_s, gidx_s, dl_s = lax.sort((binid, gidx, dloc), num_keys=1)
    bidx = jnp.arange(_NT)[:, None] * _R + jnp.minimum(jnp.arange(16)[None, :], _R)
    bounds = jnp.searchsorted(binid_s, bidx).astype(jnp.int32)
    gidx_p = jnp.concatenate([gidx_s, jnp.zeros((_CH,), jnp.int32)])
    dl_p = jnp.concatenate([dl_s, jnp.full((_CH,), _NPT, jnp.int32)])
    edata = jnp.stack([gidx_p, dl_p])

    xpad = jnp.zeros((_NPAD, _D), jnp.float32).at[:_N].set(x)
    table1 = _mm(xpad, W1a, bias1)
    h = _agg(True)(table1, edata, bounds)
    table2 = _mm(h, W2a, bias2)
    out = _agg(False)(table2, edata, bounds)
    return out[:_N]


# speedup vs baseline: 5.9367x; 1.4559x over previous
import functools

import jax
import jax.numpy as jnp
from jax import lax
from jax.experimental import pallas as pl
from jax.experimental.pallas import tpu as pltpu
from jax.experimental.pallas import tpu_sc as plsc

_N = 10000
_E = 320000
_R = 8
_D = 128
_NB = 4
_NC = 2
_NS = 16
_NT = _NC * _NS
_NPT = 320
_NPAD = _NT * _NPT
_CH = 128
_RB = 2560
_NEG = -3.0e38
_NEGH = -1.0e38


def _mm_body(x_ref, w_ref, b_ref, o_ref):
    r = pl.program_id(0)
    y = jnp.dot(x_ref[...], w_ref[0], preferred_element_type=jnp.float32)
    o_ref[0] = y + jnp.where(r == _R, b_ref[...], 0.0)


def _mm(xp, W, bias):
    out = pl.pallas_call(
        _mm_body,
        grid=(_R + 1, _NPAD // _RB),
        in_specs=[
            pl.BlockSpec((_RB, _D), lambda r, i: (i, 0)),
            pl.BlockSpec((1, _D, _D), lambda r, i: (r, 0, 0)),
            pl.BlockSpec((1, _D), lambda r, i: (0, 0)),
        ],
        out_specs=pl.BlockSpec((1, _RB, _D), lambda r, i: (r, i, 0)),
        out_shape=jax.ShapeDtypeStruct((_R + 1, _NPAD, _D), jnp.float32),
    )(xp, W, bias.reshape(1, _D))
    return out.reshape((_R + 1) * _NPAD, _D)


def _agg_body(do_relu, table, edata, bounds, out,
              bnd_v, ebuf, rows, sacc, bnd_s, semE, semR0, semR1):
    ci_ = lax.axis_index("c")
    si_ = lax.axis_index("s")
    T = si_ * _NC + ci_

    pltpu.sync_copy(bounds.at[T], bnd_v)
    bv = bnd_v[...]
    for i in range(9):
        bnd_s[i] = bv[i]

    pltpu.sync_copy(table.at[pl.ds(_R * _NPAD + T * _NPT, _NPT)],
                    sacc.at[pl.ds(0, _NPT)])

    negs16 = jnp.full((16,), _NEG, jnp.float32)
    iota16 = lax.iota(jnp.int32, 16)

    def flush(cur, regs):
        for sl in range(8):
            a = regs[sl]
            prev = sacc[cur, pl.ds(sl * 16, 16)]
            sacc[cur, pl.ds(sl * 16, 16)] = prev + jnp.where(a > _NEGH, a, 0.0)

    def rel_body(r, carry):
        start = bnd_s[r]
        end = bnd_s[r + 1]
        astart = (start // _CH) * _CH
        nch = (end - astart + _CH - 1) // _CH

        def esrc(ci):
            return edata.at[:, pl.ds(astart + ci * _CH, _CH)]

        def estart(ci):
            pltpu.async_copy(esrc(ci), ebuf.at[lax.rem(ci, 3)], semE)

        def ewait(ci):
            pltpu.make_async_copy(esrc(ci), ebuf.at[lax.rem(ci, 3)], semE).wait()

        def gstart(ci):
            e3 = lax.rem(ci, 3)
            p = lax.rem(ci, 2)

            @pl.when(p == 0)
            def _():
                pltpu.async_copy(table.at[ebuf.at[e3, 0]], rows.at[0], semR0)

            @pl.when(p == 1)
            def _():
                pltpu.async_copy(table.at[ebuf.at[e3, 0]], rows.at[1], semR1)

        def gwait(ci):
            e3 = lax.rem(ci, 3)
            p = lax.rem(ci, 2)

            @pl.when(p == 0)
            def _():
                pltpu.make_async_copy(table.at[ebuf.at[e3, 0]], rows.at[0], semR0).wait()

            @pl.when(p == 1)
            def _():
                pltpu.make_async_copy(table.at[ebuf.at[e3, 0]], rows.at[1], semR1).wait()

        @pl.when(nch > 0)
        def _():
            estart(0)
            ewait(0)
            gstart(0)

        @pl.when(nch > 1)
        def _():
            estart(1)

        def chunk(ci, car):
            p = lax.rem(ci, 2)
            e3 = lax.rem(ci, 3)

            @pl.when(ci + 1 < nch)
            def _():
                ewait(ci + 1)
                gstart(ci + 1)

            @pl.when(ci + 2 < nch)
            def _():
                estart(ci + 2)

            gwait(ci)
            base = astart + ci * _CH

            def grp(g, gc):
                cur, regs = gc
                off = g * 16
                pos = base + off + iota16
                dlv = ebuf[e3, 1, pl.ds(off, 16)]
                ok = (pos >= start) & (pos < end)
                dlm = jnp.where(ok, dlv, _NPT)
                for lane in range(16):
                    d = dlm[lane]
                    changed = d != cur

                    @pl.when(changed)
                    def _(cur=cur, regs=regs):
                        flush(cur, regs)

                    e = off + lane
                    new_regs = []
                    for sl in range(8):
                        msg = rows[p, e, pl.ds(sl * 16, 16)]
                        rg = jnp.where(changed, negs16, regs[sl])
                        new_regs.append(jnp.maximum(rg, msg))
                    regs = tuple(new_regs)
                    cur = jnp.where(changed, d, cur)
                return (cur, regs)

            return lax.fori_loop(0, _CH // 16, grp, car)

        carry0 = (jnp.int32(_NPT), tuple(negs16 for _ in range(8)))
        cur, regs = lax.fori_loop(0, nch, chunk, carry0)
        flush(cur, regs)
        return carry

    lax.fori_loop(0, _R, rel_body, 0)

    if do_relu:
        def rrow(row, cc):
            for sl in range(8):
                v = sacc[row, pl.ds(sl * 16, 16)]
                sacc[row, pl.ds(sl * 16, 16)] = jnp.maximum(v, 0.0)
            return cc

        lax.fori_loop(0, _NPT, rrow, 0)

    pltpu.sync_copy(sacc.at[pl.ds(0, _NPT)], out.at[pl.ds(T * _NPT, _NPT)])


def _agg(do_relu):
    mesh = plsc.VectorSubcoreMesh(core_axis_name="c", subcore_axis_name="s")
    return pl.kernel(
        functools.partial(_agg_body, do_relu),
        out_type=jax.ShapeDtypeStruct((_NPAD, _D), jnp.float32),
        mesh=mesh,
        scratch_types=[
            pltpu.VMEM((16,), jnp.int32),
            pltpu.VMEM((3, 2, _CH), jnp.int32),
            pltpu.VMEM((2, _CH, _D), jnp.float32),
            pltpu.VMEM((_NPT + 1, _D), jnp.float32),
            pltpu.SMEM((16,), jnp.int32),
            pltpu.SemaphoreType.DMA,
            pltpu.SemaphoreType.DMA,
            pltpu.SemaphoreType.DMA,
        ],
    )


def kernel(x, edge_index, edge_type, w1, root1, bias1, comp2, basis2, root2, bias2):
    src = edge_index[0]
    dst = edge_index[1]
    et = edge_type

    W1 = jnp.zeros((_R, _NB, _D // _NB, _NB, _D // _NB), jnp.float32)
    for b in range(_NB):
        W1 = W1.at[:, b, :, b, :].set(w1[:, b])
    W1 = W1.reshape(_R, _D, _D)
    W1a = jnp.concatenate([W1, root1[None]], axis=0)
    W2 = jnp.einsum('rb,bio->rio', comp2, basis2)
    W2a = jnp.concatenate([W2, root2[None]], axis=0)

    tid = dst // _NPT
    binid = tid * _R + et
    gidx = et * _NPAD + src
    dloc = dst - tid * _NPT
    key = binid * 512 + dloc
    key_s, gidx_s, dl_s = lax.sort((key, gidx, dloc), num_keys=1)
    bidx = jnp.arange(_NT)[:, None] * _R + jnp.minimum(jnp.arange(16)[None, :], _R)
    bounds = jnp.searchsorted(key_s, bidx * 512).astype(jnp.int32)
    gidx_p = jnp.concatenate([gidx_s, jnp.zeros((_CH,), jnp.int32)])
    dl_p = jnp.concatenate([dl_s, jnp.full((_CH,), _NPT, jnp.int32)])
    edata = jnp.stack([gidx_p, dl_p])

    xpad = jnp.zeros((_NPAD, _D), jnp.float32).at[:_N].set(x)
    table1 = _mm(xpad, W1a, bias1)
    h = _agg(True)(table1, edata, bounds)
    table2 = _mm(h, W2a, bias2)
    out = _agg(False)(table2, edata, bounds)
    return out[:_N]


# speedup vs baseline: 6.9246x; 1.1664x over previous
import functools

import jax
import jax.numpy as jnp
from jax import lax
from jax.experimental import pallas as pl
from jax.experimental.pallas import tpu as pltpu
from jax.experimental.pallas import tpu_sc as plsc

_N = 10000
_E = 320000
_R = 8
_D = 128
_NB = 4
_NC = 2
_NS = 16
_NT = _NC * _NS
_NPT = 320
_NPAD = _NT * _NPT
_CH = 128
_RB = 2560
_NEG = -3.0e38
_NEGH = -1.0e38
_KJUNK = _NPT


def _mm_body(x_ref, w_ref, b_ref, o_ref):
    r = pl.program_id(0)
    y = jnp.dot(x_ref[...], w_ref[0], preferred_element_type=jnp.float32)
    o_ref[0] = y + jnp.where(r == _R, b_ref[...], 0.0)


def _mm(xp, W, bias):
    out = pl.pallas_call(
        _mm_body,
        grid=(_R + 1, _NPAD // _RB),
        in_specs=[
            pl.BlockSpec((_RB, _D), lambda r, i: (i, 0)),
            pl.BlockSpec((1, _D, _D), lambda r, i: (r, 0, 0)),
            pl.BlockSpec((1, _D), lambda r, i: (0, 0)),
        ],
        out_specs=pl.BlockSpec((1, _RB, _D), lambda r, i: (r, i, 0)),
        out_shape=jax.ShapeDtypeStruct((_R + 1, _NPAD, _D), jnp.float32),
    )(xp, W, bias.reshape(1, _D))
    return out.reshape((_R + 1) * _NPAD, _D)


def _agg_body(do_relu, table, edata, bounds, out,
              bnd_v, ebuf, rows, sacc, semE, semR0, semR1, semR2):
    ci_ = lax.axis_index("c")
    si_ = lax.axis_index("s")
    T = si_ * _NC + ci_

    pltpu.sync_copy(bounds.at[T], bnd_v)
    bv = bnd_v[...]
    start = bv[0]
    end = bv[8]

    pltpu.sync_copy(table.at[pl.ds(_R * _NPAD + T * _NPT, _NPT)],
                    sacc.at[pl.ds(0, _NPT)])

    negs16 = jnp.full((16,), _NEG, jnp.float32)
    iota16 = lax.iota(jnp.int32, 16)

    def flush(cur, regs):
        row = jnp.bitwise_and(cur, 511)
        for sl in range(8):
            a = regs[sl]
            prev = sacc[row, pl.ds(sl * 16, 16)]
            sacc[row, pl.ds(sl * 16, 16)] = prev + jnp.where(a > _NEGH, a, 0.0)

    astart = (start // _CH) * _CH
    nch = (end - astart + _CH - 1) // _CH

    def esrc(ci):
        return edata.at[:, pl.ds(astart + ci * _CH, _CH)]

    def estart(ci):
        pltpu.async_copy(esrc(ci), ebuf.at[lax.rem(ci, 4)], semE)

    def ewait(ci):
        pltpu.make_async_copy(esrc(ci), ebuf.at[lax.rem(ci, 4)], semE).wait()

    def gstart(ci):
        e4 = lax.rem(ci, 4)
        p = lax.rem(ci, 3)

        @pl.when(p == 0)
        def _():
            pltpu.async_copy(table.at[ebuf.at[e4, 0]], rows.at[0], semR0)

        @pl.when(p == 1)
        def _():
            pltpu.async_copy(table.at[ebuf.at[e4, 0]], rows.at[1], semR1)

        @pl.when(p == 2)
        def _():
            pltpu.async_copy(table.at[ebuf.at[e4, 0]], rows.at[2], semR2)

    def gwait(ci):
        e4 = lax.rem(ci, 4)
        p = lax.rem(ci, 3)

        @pl.when(p == 0)
        def _():
            pltpu.make_async_copy(table.at[ebuf.at[e4, 0]], rows.at[0], semR0).wait()

        @pl.when(p == 1)
        def _():
            pltpu.make_async_copy(table.at[ebuf.at[e4, 0]], rows.at[1], semR1).wait()

        @pl.when(p == 2)
        def _():
            pltpu.make_async_copy(table.at[ebuf.at[e4, 0]], rows.at[2], semR2).wait()

    @pl.when(nch > 0)
    def _():
        estart(0)
        ewait(0)
        gstart(0)

    @pl.when(nch > 1)
    def _():
        estart(1)
        ewait(1)
        gstart(1)

    @pl.when(nch > 2)
    def _():
        estart(2)

    def chunk(ci, car):
        p = lax.rem(ci, 3)
        e4 = lax.rem(ci, 4)

        @pl.when(ci + 2 < nch)
        def _():
            ewait(ci + 2)
            gstart(ci + 2)

        @pl.when(ci + 3 < nch)
        def _():
            estart(ci + 3)

        gwait(ci)
        base = astart + ci * _CH

        def grp(g, gc):
            cur, regs = gc
            off = g * 16
            pos = base + off + iota16
            kv = ebuf[e4, 1, pl.ds(off, 16)]
            ok = (pos >= start) & (pos < end)
            km = jnp.where(ok, kv, _KJUNK)
            for lane in range(16):
                k = km[lane]
                changed = k != cur

                @pl.when(changed)
                def _(cur=cur, regs=regs):
                    flush(cur, regs)

                e = off + lane
                new_regs = []
                for sl in range(8):
                    msg = rows[p, e, pl.ds(sl * 16, 16)]
                    rg = jnp.where(changed, negs16, regs[sl])
                    new_regs.append(jnp.maximum(rg, msg))
                regs = tuple(new_regs)
                cur = jnp.where(changed, k, cur)
            return (cur, regs)

        return lax.fori_loop(0, _CH // 16, grp, car)

    carry0 = (jnp.int32(_KJUNK), tuple(negs16 for _ in range(8)))
    cur, regs = lax.fori_loop(0, nch, chunk, carry0)
    flush(cur, regs)

    if do_relu:
        def rrow(row, cc):
            for sl in range(8):
                v = sacc[row, pl.ds(sl * 16, 16)]
                sacc[row, pl.ds(sl * 16, 16)] = jnp.maximum(v, 0.0)
            return cc

        lax.fori_loop(0, _NPT, rrow, 0)

    pltpu.sync_copy(sacc.at[pl.ds(0, _NPT)], out.at[pl.ds(T * _NPT, _NPT)])


def _agg(do_relu):
    mesh = plsc.VectorSubcoreMesh(core_axis_name="c", subcore_axis_name="s")
    return pl.kernel(
        functools.partial(_agg_body, do_relu),
        out_type=jax.ShapeDtypeStruct((_NPAD, _D), jnp.float32),
        mesh=mesh,
        scratch_types=[
            pltpu.VMEM((16,), jnp.int32),
            pltpu.VMEM((4, 2, _CH), jnp.int32),
            pltpu.VMEM((3, _CH, _D), jnp.float32),
            pltpu.VMEM((_NPT + 1, _D), jnp.float32),
            pltpu.SemaphoreType.DMA,
            pltpu.SemaphoreType.DMA,
            pltpu.SemaphoreType.DMA,
            pltpu.SemaphoreType.DMA,
        ],
    )


def kernel(x, edge_index, edge_type, w1, root1, bias1, comp2, basis2, root2, bias2):
    src = edge_index[0]
    dst = edge_index[1]
    et = edge_type

    W1 = jnp.zeros((_R, _NB, _D // _NB, _NB, _D // _NB), jnp.float32)
    for b in range(_NB):
        W1 = W1.at[:, b, :, b, :].set(w1[:, b])
    W1 = W1.reshape(_R, _D, _D)
    W1a = jnp.concatenate([W1, root1[None]], axis=0)
    W2 = jnp.einsum('rb,bio->rio', comp2, basis2)
    W2a = jnp.concatenate([W2, root2[None]], axis=0)

    tid = dst // _NPT
    dloc = dst - tid * _NPT
    key = (tid * _R + et) * 512 + dloc
    gidx = et * _NPAD + src
    key_s, gidx_s = lax.sort((key, gidx), num_keys=1)
    bidx = jnp.arange(_NT)[:, None] * _R + jnp.minimum(jnp.arange(16)[None, :], _R)
    bounds = jnp.searchsorted(key_s, bidx * 512).astype(jnp.int32)
    gidx_p = jnp.concatenate([gidx_s, jnp.zeros((_CH,), jnp.int32)])
    key_p = jnp.concatenate([key_s, jnp.full((_CH,), _KJUNK, jnp.int32)])
    edata = jnp.stack([gidx_p, key_p])

    xpad = jnp.zeros((_NPAD, _D), jnp.float32).at[:_N].set(x)
    table1 = _mm(xpad, W1a, bias1)
    h = _agg(True)(table1, edata, bounds)
    table2 = _mm(h, W2a, bias2)
    out = _agg(False)(table2, edata, bounds)
    return out[:_N]


# speedup vs baseline: 7.1960x; 1.0392x over previous
import functools

import jax
import jax.numpy as jnp
from jax import lax
from jax.experimental import pallas as pl
from jax.experimental.pallas import tpu as pltpu
from jax.experimental.pallas import tpu_sc as plsc

_N = 10000
_E = 320000
_R = 8
_D = 128
_NB = 4
_NC = 2
_NS = 16
_NT = _NC * _NS
_NPT = 320
_NPAD = _NT * _NPT
_CH = 128
_RB = 2560
_NEG = -3.0e38
_NEGH = -1.0e38
_KJUNK = _NPT


def _mm_body(x_ref, w_ref, b_ref, o_ref):
    r = pl.program_id(0)
    y = jnp.dot(x_ref[...], w_ref[0], preferred_element_type=jnp.float32)
    o_ref[0] = y + jnp.where(r == _R, b_ref[...], 0.0)


def _mm(xp, W, bias):
    out = pl.pallas_call(
        _mm_body,
        grid=(_R + 1, _NPAD // _RB),
        in_specs=[
            pl.BlockSpec((_RB, _D), lambda r, i: (i, 0)),
            pl.BlockSpec((1, _D, _D), lambda r, i: (r, 0, 0)),
            pl.BlockSpec((1, _D), lambda r, i: (0, 0)),
        ],
        out_specs=pl.BlockSpec((1, _RB, _D), lambda r, i: (r, i, 0)),
        out_shape=jax.ShapeDtypeStruct((_R + 1, _NPAD, _D), jnp.float32),
    )(xp, W, bias.reshape(1, _D))
    return out.reshape((_R + 1) * _NPAD, _D)


def _agg_body(do_relu, table, edata, bounds, out,
              bnd_v, ebuf, idxb, rows, sacc, semE, semR0, semR1, semR2):
    ci_ = lax.axis_index("c")
    si_ = lax.axis_index("s")
    T = si_ * _NC + ci_

    pltpu.sync_copy(bounds.at[T], bnd_v)
    bv = bnd_v[...]
    start = bv[0]
    end = bv[8]

    pltpu.sync_copy(table.at[pl.ds(_R * _NPAD + T * _NPT, _NPT)],
                    sacc.at[pl.ds(0, _NPT)])

    negs16 = jnp.full((16,), _NEG, jnp.float32)
    iota16 = lax.iota(jnp.int32, 16)

    def flush(cur, regs):
        row = jnp.bitwise_and(cur, 511)
        for sl in range(8):
            a = regs[sl]
            prev = sacc[row, pl.ds(sl * 16, 16)]
            sacc[row, pl.ds(sl * 16, 16)] = prev + jnp.where(a > _NEGH, a, 0.0)

    astart = (start // _CH) * _CH
    nch = (end - astart + _CH - 1) // _CH

    def esrc(ci):
        return edata.at[pl.ds(astart + ci * _CH, _CH)]

    def estart(ci):
        pltpu.async_copy(esrc(ci), ebuf.at[lax.rem(ci, 4)], semE)

    def ewait(ci):
        pltpu.make_async_copy(esrc(ci), ebuf.at[lax.rem(ci, 4)], semE).wait()

    def eprep(ci):
        e4 = lax.rem(ci, 4)

        def pg(g, cc):
            off = g * 16
            pk = ebuf[e4, pl.ds(off, 16)]
            srcv = jnp.bitwise_and(pk, 16383)
            etv = jnp.bitwise_and(lax.shift_right_logical(pk, 23), 7)
            idxb[e4, pl.ds(off, 16)] = etv * _NPAD + srcv
            return cc

        lax.fori_loop(0, _CH // 16, pg, 0)

    def gstart(ci):
        e4 = lax.rem(ci, 4)
        p = lax.rem(ci, 3)

        @pl.when(p == 0)
        def _():
            pltpu.async_copy(table.at[idxb.at[e4]], rows.at[0], semR0)

        @pl.when(p == 1)
        def _():
            pltpu.async_copy(table.at[idxb.at[e4]], rows.at[1], semR1)

        @pl.when(p == 2)
        def _():
            pltpu.async_copy(table.at[idxb.at[e4]], rows.at[2], semR2)

    def gwait(ci):
        e4 = lax.rem(ci, 4)
        p = lax.rem(ci, 3)

        @pl.when(p == 0)
        def _():
            pltpu.make_async_copy(table.at[idxb.at[e4]], rows.at[0], semR0).wait()

        @pl.when(p == 1)
        def _():
            pltpu.make_async_copy(table.at[idxb.at[e4]], rows.at[1], semR1).wait()

        @pl.when(p == 2)
        def _():
            pltpu.make_async_copy(table.at[idxb.at[e4]], rows.at[2], semR2).wait()

    @pl.when(nch > 0)
    def _():
        estart(0)
        ewait(0)
        eprep(0)
        gstart(0)

    @pl.when(nch > 1)
    def _():
        estart(1)
        ewait(1)
        eprep(1)
        gstart(1)

    @pl.when(nch > 2)
    def _():
        estart(2)

    def chunk(ci, car):
        p = lax.rem(ci, 3)
        e4 = lax.rem(ci, 4)

        @pl.when(ci + 2 < nch)
        def _():
            ewait(ci + 2)
            eprep(ci + 2)
            gstart(ci + 2)

        @pl.when(ci + 3 < nch)
        def _():
            estart(ci + 3)

        gwait(ci)
        base = astart + ci * _CH

        def grp(g, gc):
            cur, regs = gc
            off = g * 16
            pos = base + off + iota16
            kv = lax.shift_right_logical(ebuf[e4, pl.ds(off, 16)], 14)
            ok = (pos >= start) & (pos < end)
            km = jnp.where(ok, kv, _KJUNK)
            for lane in range(16):
                k = km[lane]
                changed = k != cur

                @pl.when(changed)
                def _(cur=cur, regs=regs):
                    flush(cur, regs)

                e = off + lane
                new_regs = []
                for sl in range(8):
                    msg = rows[p, e, pl.ds(sl * 16, 16)]
                    rg = jnp.where(changed, negs16, regs[sl])
                    new_regs.append(jnp.maximum(rg, msg))
                regs = tuple(new_regs)
                cur = jnp.where(changed, k, cur)
            return (cur, regs)

        return lax.fori_loop(0, _CH // 16, grp, car)

    carry0 = (jnp.int32(_KJUNK), tuple(negs16 for _ in range(8)))
    cur, regs = lax.fori_loop(0, nch, chunk, carry0)
    flush(cur, regs)

    if do_relu:
        def rrow(row, cc):
            for sl in range(8):
                v = sacc[row, pl.ds(sl * 16, 16)]
                sacc[row, pl.ds(sl * 16, 16)] = jnp.maximum(v, 0.0)
            return cc

        lax.fori_loop(0, _NPT, rrow, 0)

    pltpu.sync_copy(sacc.at[pl.ds(0, _NPT)], out.at[pl.ds(T * _NPT, _NPT)])


def _agg(do_relu):
    mesh = plsc.VectorSubcoreMesh(core_axis_name="c", subcore_axis_name="s")
    return pl.kernel(
        functools.partial(_agg_body, do_relu),
        out_type=jax.ShapeDtypeStruct((_NPAD, _D), jnp.float32),
        mesh=mesh,
        scratch_types=[
            pltpu.VMEM((16,), jnp.int32),
            pltpu.VMEM((4, _CH), jnp.int32),
            pltpu.VMEM((4, _CH), jnp.int32),
            pltpu.VMEM((3, _CH, _D), jnp.float32),
            pltpu.VMEM((_NPT + 1, _D), jnp.float32),
            pltpu.SemaphoreType.DMA,
            pltpu.SemaphoreType.DMA,
            pltpu.SemaphoreType.DMA,
            pltpu.SemaphoreType.DMA,
        ],
    )


def kernel(x, edge_index, edge_type, w1, root1, bias1, comp2, basis2, root2, bias2):
    src = edge_index[0]
    dst = edge_index[1]
    et = edge_type

    W1 = jnp.zeros((_R, _NB, _D // _NB, _NB, _D // _NB), jnp.float32)
    for b in range(_NB):
        W1 = W1.at[:, b, :, b, :].set(w1[:, b])
    W1 = W1.reshape(_R, _D, _D)
    W1a = jnp.concatenate([W1, root1[None]], axis=0)
    W2 = jnp.einsum('rb,bio->rio', comp2, basis2)
    W2a = jnp.concatenate([W2, root2[None]], axis=0)

    tid = dst // _NPT
    dloc = dst - tid * _NPT
    key = (tid * _R + et) * 512 + dloc
    packed = key * 16384 + src
    packed_s = lax.sort(packed)
    bidx = jnp.arange(_NT)[:, None] * _R + jnp.minimum(jnp.arange(16)[None, :], _R)
    q = bidx * 512
    queries = jnp.where(q >= 131072, jnp.int32(2**31 - 1), q * 16384)
    bounds = jnp.searchsorted(packed_s, queries).astype(jnp.int32)
    edata = jnp.concatenate([packed_s, jnp.zeros((_CH,), jnp.int32)])

    xpad = jnp.zeros((_NPAD, _D), jnp.float32).at[:_N].set(x)
    table1 = _mm(xpad, W1a, bias1)
    h = _agg(True)(table1, edata, bounds)
    table2 = _mm(h, W2a, bias2)
    out = _agg(False)(table2, edata, bounds)
    return out[:_N]


# speedup vs baseline: 9.7676x; 1.3574x over previous
import functools

import jax
import jax.numpy as jnp
from jax import lax
from jax.experimental import pallas as pl
from jax.experimental.pallas import tpu as pltpu
from jax.experimental.pallas import tpu_sc as plsc

_N = 10000
_E = 320000
_R = 8
_D = 128
_NB = 4
_NC = 2
_NS = 16
_NT = _NC * _NS
_NPT = 320
_NPAD = _NT * _NPT
_CH = 128
_RB = 2560
_NEG = -3.0e38
_NEGH = -1.0e38
_KJUNK = _NPT


def _mm_body(x_ref, w_ref, b_ref, o_ref):
    r = pl.program_id(0)
    y = jnp.dot(x_ref[...], w_ref[0], preferred_element_type=jnp.float32)
    o_ref[0] = y + jnp.where(r == _R, b_ref[...], 0.0)


def _mm(xp, W, bias):
    out = pl.pallas_call(
        _mm_body,
        grid=(_R + 1, _NPAD // _RB),
        in_specs=[
            pl.BlockSpec((_RB, _D), lambda r, i: (i, 0)),
            pl.BlockSpec((1, _D, _D), lambda r, i: (r, 0, 0)),
            pl.BlockSpec((1, _D), lambda r, i: (0, 0)),
        ],
        out_specs=pl.BlockSpec((1, _RB, _D), lambda r, i: (r, i, 0)),
        out_shape=jax.ShapeDtypeStruct((_R + 1, _NPAD, _D), jnp.float32),
    )(xp, W, bias.reshape(1, _D))
    return out.reshape((_R + 1) * _NPAD, _D)


def _agg_body(do_relu, table, edata, bounds, out,
              bnd_v, ebuf, idxb, rows, sacc, semE, semR0, semR1, semR2):
    ci_ = lax.axis_index("c")
    si_ = lax.axis_index("s")
    T = si_ * _NC + ci_

    pltpu.sync_copy(bounds.at[T], bnd_v)
    bv = bnd_v[...]
    start = bv[0]
    end = bv[8]

    pltpu.sync_copy(table.at[pl.ds(_R * _NPAD + T * _NPT, _NPT)],
                    sacc.at[pl.ds(0, _NPT)])

    negs16 = jnp.full((16,), _NEG, jnp.float32)
    iota16 = lax.iota(jnp.int32, 16)

    def flush(cur, regs):
        row = jnp.bitwise_and(cur, 511)
        for sl in range(8):
            a = regs[sl]
            prev = sacc[row, pl.ds(sl * 16, 16)]
            sacc[row, pl.ds(sl * 16, 16)] = prev + jnp.where(a > _NEGH, a, 0.0)

    astart = (start // _CH) * _CH
    nch = (end - astart + _CH - 1) // _CH

    def esrc(ci):
        return edata.at[pl.ds(astart + ci * _CH, _CH)]

    def estart(ci):
        pltpu.async_copy(esrc(ci), ebuf.at[lax.rem(ci, 4)], semE)

    def ewait(ci):
        pltpu.make_async_copy(esrc(ci), ebuf.at[lax.rem(ci, 4)], semE).wait()

    def eprep(ci):
        e4 = lax.rem(ci, 4)

        def pg(g, cc):
            off = g * 16
            pk = ebuf[e4, pl.ds(off, 16)]
            srcv = jnp.bitwise_and(pk, 16383)
            etv = jnp.bitwise_and(lax.shift_right_logical(pk, 23), 7)
            idxb[e4, pl.ds(off, 16)] = etv * _NPAD + srcv
            return cc

        lax.fori_loop(0, _CH // 16, pg, 0)

    def gstart(ci):
        e4 = lax.rem(ci, 4)
        p = lax.rem(ci, 3)

        @pl.when(p == 0)
        def _():
            pltpu.async_copy(table.at[idxb.at[e4]], rows.at[0], semR0)

        @pl.when(p == 1)
        def _():
            pltpu.async_copy(table.at[idxb.at[e4]], rows.at[1], semR1)

        @pl.when(p == 2)
        def _():
            pltpu.async_copy(table.at[idxb.at[e4]], rows.at[2], semR2)

    def gwait(ci):
        e4 = lax.rem(ci, 4)
        p = lax.rem(ci, 3)

        @pl.when(p == 0)
        def _():
            pltpu.make_async_copy(table.at[idxb.at[e4]], rows.at[0], semR0).wait()

        @pl.when(p == 1)
        def _():
            pltpu.make_async_copy(table.at[idxb.at[e4]], rows.at[1], semR1).wait()

        @pl.when(p == 2)
        def _():
            pltpu.make_async_copy(table.at[idxb.at[e4]], rows.at[2], semR2).wait()

    @pl.when(nch > 0)
    def _():
        estart(0)
        ewait(0)
        eprep(0)
        gstart(0)

    @pl.when(nch > 1)
    def _():
        estart(1)
        ewait(1)
        eprep(1)
        gstart(1)

    @pl.when(nch > 2)
    def _():
        estart(2)

    def chunk(ci, car):
        p = lax.rem(ci, 3)
        e4 = lax.rem(ci, 4)

        @pl.when(ci + 2 < nch)
        def _():
            ewait(ci + 2)
            eprep(ci + 2)
            gstart(ci + 2)

        @pl.when(ci + 3 < nch)
        def _():
            estart(ci + 3)

        gwait(ci)
        base = astart + ci * _CH

        def grp(g, gc):
            cur, regs = gc
            off = g * 16
            pos = base + off + iota16
            kv = lax.shift_right_logical(ebuf[e4, pl.ds(off, 16)], 14)
            ok = (pos >= start) & (pos < end)
            km = jnp.where(ok, kv, _KJUNK)
            for lane in range(16):
                k = km[lane]
                changed = k != cur

                @pl.when(changed)
                def _(cur=cur, regs=regs):
                    flush(cur, regs)

                e = off + lane
                new_regs = []
                for sl in range(8):
                    msg = rows[p, e, pl.ds(sl * 16, 16)]
                    rg = jnp.where(changed, negs16, regs[sl])
                    new_regs.append(jnp.maximum(rg, msg))
                regs = tuple(new_regs)
                cur = jnp.where(changed, k, cur)
            return (cur, regs)

        return lax.fori_loop(0, _CH // 16, grp, car)

    carry0 = (jnp.int32(_KJUNK), tuple(negs16 for _ in range(8)))
    cur, regs = lax.fori_loop(0, nch, chunk, carry0)
    flush(cur, regs)

    if do_relu:
        def rrow(row, cc):
            for sl in range(8):
                v = sacc[row, pl.ds(sl * 16, 16)]
                sacc[row, pl.ds(sl * 16, 16)] = jnp.maximum(v, 0.0)
            return cc

        lax.fori_loop(0, _NPT, rrow, 0)

    pltpu.sync_copy(sacc.at[pl.ds(0, _NPT)], out.at[pl.ds(T * _NPT, _NPT)])


def _agg(do_relu):
    mesh = plsc.VectorSubcoreMesh(core_axis_name="c", subcore_axis_name="s")
    return pl.kernel(
        functools.partial(_agg_body, do_relu),
        out_type=jax.ShapeDtypeStruct((_NPAD, _D), jnp.float32),
        mesh=mesh,
        scratch_types=[
            pltpu.VMEM((16,), jnp.int32),
            pltpu.VMEM((4, _CH), jnp.int32),
            pltpu.VMEM((4, _CH), jnp.int32),
            pltpu.VMEM((3, _CH, _D), jnp.float32),
            pltpu.VMEM((_NPT + 1, _D), jnp.float32),
            pltpu.SemaphoreType.DMA,
            pltpu.SemaphoreType.DMA,
            pltpu.SemaphoreType.DMA,
            pltpu.SemaphoreType.DMA,
        ],
    )


def kernel(x, edge_index, edge_type, w1, root1, bias1, comp2, basis2, root2, bias2):
    src = edge_index[0]
    dst = edge_index[1]
    et = edge_type

    W1 = jnp.zeros((_R, _NB, _D // _NB, _NB, _D // _NB), jnp.float32)
    for b in range(_NB):
        W1 = W1.at[:, b, :, b, :].set(w1[:, b])
    W1 = W1.reshape(_R, _D, _D)
    W1a = jnp.concatenate([W1, root1[None]], axis=0)
    W2 = jnp.einsum('rb,bio->rio', comp2, basis2)
    W2a = jnp.concatenate([W2, root2[None]], axis=0)

    tid = dst // _NPT
    dloc = dst - tid * _NPT
    key = (tid * _R + et) * 512 + dloc
    packed = key * 16384 + src
    packed_s = lax.sort(packed, is_stable=False)
    bidx = jnp.arange(_NT)[:, None] * _R + jnp.minimum(jnp.arange(16)[None, :], _R)
    q = bidx * 512
    queries = jnp.where(q >= 131072, jnp.int32(2**31 - 1), q * 16384)
    bounds = jnp.searchsorted(packed_s, queries).astype(jnp.int32)
    edata = jnp.concatenate([packed_s, jnp.zeros((_CH,), jnp.int32)])

    xpad = jnp.zeros((_NPAD, _D), jnp.float32).at[:_N].set(x)
    table1 = _mm(xpad, W1a, bias1)
    h = _agg(True)(table1, edata, bounds)
    table2 = _mm(h, W2a, bias2)
    out = _agg(False)(table2, edata, bounds)
    return out[:_N]
